# Initial kernel scaffold; baseline (speedup 1.0000x reference)
import jax
import jax.numpy as jnp
from jax.experimental import pallas as pl


def kernel(x, edge_index, batch, num_graphs, W0, conv_weight, W_ih, W_hh, b_ih, b_hh, w1, b1, w2, b2):
    raise NotImplementedError("write your pallas kernel here")


# speedup vs baseline: 2.5715x; 2.5715x over previous
import functools

import jax
import jax.numpy as jnp
from jax import lax
from jax.experimental import pallas as pl
from jax.experimental.pallas import tpu as pltpu
from jax.experimental.pallas import tpu_sc as plsc

_N = 10000
_D = 128
_H = 128
_G = 64
_RB = 400
_NRB = _N // _RB

_NTILES = 32
_CH = 128
_NCH = 80
_EPT = _CH * _NCH
_EPAD = _EPT * _NTILES
_NPAD = 10240
_ZR = 64
_RPT = _NPAD // 16


def _init_body(x_ref, w0_ref, wc_ref, x1_ref, m_ref):
    x1 = jax.nn.sigmoid(
        jnp.dot(x_ref[...], w0_ref[...], preferred_element_type=jnp.float32))
    x1_ref[...] = x1
    m_ref[...] = jnp.dot(x1, wc_ref[...], preferred_element_type=jnp.float32)


_init_call = pl.pallas_call(
    _init_body,
    grid=(_NRB,),
    in_specs=[
        pl.BlockSpec((_RB, _D), lambda i: (i, 0)),
        pl.BlockSpec((_D, _H), lambda i: (0, 0)),
        pl.BlockSpec((_H, _H), lambda i: (0, 0)),
    ],
    out_specs=[
        pl.BlockSpec((_RB, _H), lambda i: (i, 0)),
        pl.BlockSpec((_RB, _H), lambda i: (i, 0)),
    ],
    out_shape=[
        jax.ShapeDtypeStruct((_N, _H), jnp.float32),
        jax.ShapeDtypeStruct((_N, _H), jnp.float32),
    ],
)


def _gru_common(parts_ref, h_ref, wih_ref, whh_ref, bih_ref, bhh_ref):
    agg = parts_ref[0] + parts_ref[1]
    h = h_ref[...]
    gi = jnp.dot(agg, wih_ref[...], preferred_element_type=jnp.float32) + bih_ref[...]
    gh = jnp.dot(h, whh_ref[...], preferred_element_type=jnp.float32) + bhh_ref[...]
    r = jax.nn.sigmoid(gi[:, :_H] + gh[:, :_H])
    z = jax.nn.sigmoid(gi[:, _H:2 * _H] + gh[:, _H:2 * _H])
    n = jnp.tanh(gi[:, 2 * _H:] + r * gh[:, 2 * _H:])
    return (1.0 - z) * n + z * h


def _gru_mid_body(parts_ref, h_ref, wih_ref, whh_ref, bih_ref, bhh_ref, wc_ref,
                  h_out_ref, m_out_ref):
    hn = _gru_common(parts_ref, h_ref, wih_ref, whh_ref, bih_ref, bhh_ref)
    h_out_ref[...] = hn
    m_out_ref[...] = jnp.dot(hn, wc_ref[...], preferred_element_type=jnp.float32)


_gru_mid_call = pl.pallas_call(
    _gru_mid_body,
    grid=(_NRB,),
    in_specs=[
        pl.BlockSpec((2, _RB, _H), lambda i: (0, i, 0)),
        pl.BlockSpec((_RB, _H), lambda i: (i, 0)),
        pl.BlockSpec((_H, 3 * _H), lambda i: (0, 0)),
        pl.BlockSpec((_H, 3 * _H), lambda i: (0, 0)),
        pl.BlockSpec((1, 3 * _H), lambda i: (0, 0)),
        pl.BlockSpec((1, 3 * _H), lambda i: (0, 0)),
        pl.BlockSpec((_H, _H), lambda i: (0, 0)),
    ],
    out_specs=[
        pl.BlockSpec((_RB, _H), lambda i: (i, 0)),
        pl.BlockSpec((_RB, _H), lambda i: (i, 0)),
    ],
    out_shape=[
        jax.ShapeDtypeStruct((_N, _H), jnp.float32),
        jax.ShapeDtypeStruct((_N, _H), jnp.float32),
    ],
)


def _gru_head_body(parts_ref, h_ref, wih_ref, whh_ref, bih_ref, bhh_ref,
                   wms_ref, bms_ref, ms_ref):
    hn = _gru_common(parts_ref, h_ref, wih_ref, whh_ref, bih_ref, bhh_ref)
    xo = jnp.maximum(hn, 0.0)
    ms = jnp.dot(xo, wms_ref[...], preferred_element_type=jnp.float32) + bms_ref[...]
    mu = ms[:, 0:1]
    s = ms[:, 1:2]
    sp = jnp.maximum(s, 0.0) + jnp.log1p(jnp.exp(-jnp.abs(s)))
    ms_ref[...] = jnp.concatenate([mu, sp], axis=1)


_gru_head_call = pl.pallas_call(
    _gru_head_body,
    grid=(_NRB,),
    in_specs=[
        pl.BlockSpec((2, _RB, _H), lambda i: (0, i, 0)),
        pl.BlockSpec((_RB, _H), lambda i: (i, 0)),
        pl.BlockSpec((_H, 3 * _H), lambda i: (0, 0)),
        pl.BlockSpec((_H, 3 * _H), lambda i: (0, 0)),
        pl.BlockSpec((1, 3 * _H), lambda i: (0, 0)),
        pl.BlockSpec((1, 3 * _H), lambda i: (0, 0)),
        pl.BlockSpec((_H, 2), lambda i: (0, 0)),
        pl.BlockSpec((1, 2), lambda i: (0, 0)),
    ],
    out_specs=pl.BlockSpec((_RB, 2), lambda i: (i, 0)),
    out_shape=jax.ShapeDtypeStruct((_N, 2), jnp.float32),
)


def _sums_body(ms_ref, b_ref, out_ref):
    i = pl.program_id(0)

    @pl.when(i == 0)
    def _():
        out_ref[...] = jnp.zeros_like(out_ref)

    b = b_ref[:, 0]
    onehot = (b[None, :] == lax.broadcasted_iota(jnp.int32, (_G, _RB), 0).astype(jnp.float32))
    out_ref[...] += jnp.dot(onehot.astype(jnp.float32), ms_ref[...],
                            preferred_element_type=jnp.float32)


_sums_call = pl.pallas_call(
    _sums_body,
    grid=(_NRB,),
    in_specs=[
        pl.BlockSpec((_RB, 2), lambda i: (i, 0)),
        pl.BlockSpec((_RB, 1), lambda i: (i, 0)),
    ],
    out_specs=pl.BlockSpec((_G, 2), lambda i: (0, 0)),
    out_shape=jax.ShapeDtypeStruct((_G, 2), jnp.float32),
)


def _apply_body(ms_ref, b_ref, sums_ref, out_ref):
    b = b_ref[:, 0]
    onehot = (b[:, None] == lax.broadcasted_iota(jnp.int32, (_RB, _G), 1).astype(jnp.float32))
    gath = jnp.dot(onehot.astype(jnp.float32), sums_ref[...],
                   preferred_element_type=jnp.float32)
    mu = ms_ref[:, 0:1]
    sig = ms_ref[:, 1:2]
    out_ref[...] = mu - gath[:, 0:1] * (sig / gath[:, 1:2])


_apply_call = pl.pallas_call(
    _apply_body,
    grid=(_NRB,),
    in_specs=[
        pl.BlockSpec((_RB, 2), lambda i: (i, 0)),
        pl.BlockSpec((_RB, 1), lambda i: (i, 0)),
        pl.BlockSpec((_G, 2), lambda i: (0, 0)),
    ],
    out_specs=pl.BlockSpec((_RB, 1), lambda i: (i, 0)),
    out_shape=jax.ShapeDtypeStruct((_N, 1), jnp.float32),
)


def _segment_sum_body(m_hbm, src_hbm, dst_hbm, out_hbm,
                      src_v, dst_v, rows_v, zbuf, acc, sem):
    c = lax.axis_index("c")
    s = lax.axis_index("s")
    wid = c * 16 + s
    zero16 = jnp.zeros((16,), jnp.float32)

    def zrow(k, carry):
        zbuf[k // 8, pl.ds((k % 8) * 16, 16)] = zero16
        return carry

    lax.fori_loop(0, _ZR * 8, zrow, 0)

    rows_per_tile = _NPAD // 16

    def zslice(j, carry):
        off = pl.multiple_of(s * rows_per_tile + j * _ZR, _ZR)
        pltpu.sync_copy(zbuf, acc.at[pl.ds(off, _ZR)])
        return carry

    lax.fori_loop(0, rows_per_tile // _ZR, zslice, 0)
    plsc.subcore_barrier()

    base = wid * _EPT

    def ebody(t, carry):
        eb = pl.multiple_of(base + t * _CH, _CH)
        pltpu.sync_copy(src_hbm.at[pl.ds(eb, _CH)], src_v)
        pltpu.sync_copy(dst_hbm.at[pl.ds(eb, _CH)], dst_v)
        pltpu.async_copy(m_hbm.at[src_v], rows_v, sem).wait()
        pltpu.sync_copy(rows_v, acc.at[dst_v], add=True)
        return carry

    lax.fori_loop(0, _NCH, ebody, 0)
    plsc.subcore_barrier()

    out_off = pl.multiple_of(s * _RPT, _RPT)
    pltpu.sync_copy(acc.at[pl.ds(out_off, _RPT)],
                    out_hbm.at[c, pl.ds(out_off, _RPT)])


@functools.cache
def _make_sc_call():
    mesh = plsc.VectorSubcoreMesh(core_axis_name="c", subcore_axis_name="s")
    return pl.kernel(
        _segment_sum_body,
        out_type=jax.ShapeDtypeStruct((2, _NPAD, _H), jnp.float32),
        mesh=mesh,
        scratch_types=[
            pltpu.VMEM((_CH,), jnp.int32),
            pltpu.VMEM((_CH,), jnp.int32),
            pltpu.VMEM((_CH, _H), jnp.float32),
            pltpu.VMEM((_ZR, _H), jnp.float32),
            pltpu.VMEM_SHARED((_NPAD, _H), jnp.float32),
            pltpu.SemaphoreType.DMA,
        ],
    )


def _edge_segment_sum(m, src_p, dst_p):
    return _make_sc_call()(m, src_p, dst_p)


def kernel(x, edge_index, batch, num_graphs,
           W0, conv_weight, W_ih, W_hh, b_ih, b_hh, w1, b1, w2, b2):
    src = edge_index[0]
    dst = edge_index[1]
    e = src.shape[0]
    pad = _EPAD - e
    src_p = jnp.concatenate([src, jnp.zeros((pad,), jnp.int32)])
    dst_p = jnp.concatenate([dst, jnp.full((pad,), _N, jnp.int32)])

    bih2 = b_ih.reshape(1, 3 * _H)
    bhh2 = b_hh.reshape(1, 3 * _H)
    wms = jnp.concatenate([w1, w2], axis=1)
    bms = jnp.concatenate([b1, b2]).reshape(1, 2)
    batchf = batch.astype(jnp.float32).reshape(_N, 1)

    x1, m = _init_call(x, W0, conv_weight[0])
    h = x1
    for i in range(2):
        parts = _edge_segment_sum(m, src_p, dst_p)
        h, m = _gru_mid_call(parts, h, W_ih, W_hh, bih2, bhh2,
                             conv_weight[i + 1])
    parts = _edge_segment_sum(m, src_p, dst_p)
    musig = _gru_head_call(parts, h, W_ih, W_hh, bih2, bhh2, wms, bms)
    sums = _sums_call(musig, batchf)
    mu_c = _apply_call(musig, batchf, sums)
    return (mu_c[:, 0], x1, musig[:, 1], musig[:, 0])
